# baseline (speedup 1.0000x reference)
import functools

import jax
import jax.numpy as jnp
from jax import lax
from jax.experimental import pallas as pl
from jax.experimental.pallas import tpu as pltpu
from jax.experimental.pallas import tpu_sc as plsc

D = 64
K = 4
C = 16
L = 50
ITER = 2

_NC = 2
_NS = 16
_NW = _NC * _NS
_CHUNK = 80


def _sc_gather(emb, ids3):
    n_chunk = ids3.shape[1]
    rpw = n_chunk * _CHUNK
    total = _NW * rpw
    mesh = plsc.VectorSubcoreMesh(core_axis_name="c", subcore_axis_name="s")

    @functools.partial(
        pl.kernel,
        mesh=mesh,
        out_type=jax.ShapeDtypeStruct((total, D), jnp.float32),
        scratch_types=[
            pltpu.VMEM((n_chunk, _CHUNK), jnp.int32),
            pltpu.VMEM((rpw, D), jnp.float32),
            pltpu.SemaphoreType.DMA,
        ],
        compiler_params=pltpu.CompilerParams(use_tc_tiling_on_sc=False),
    )
    def gather_kernel(emb_hbm, idx_hbm, out_hbm, idx_v, rows_v, sem):
        wid = lax.axis_index("s") * _NC + lax.axis_index("c")
        pltpu.sync_copy(idx_hbm.at[wid], idx_v)
        copies = []
        for c in range(n_chunk):
            copies.append(
                pltpu.async_copy(
                    emb_hbm.at[idx_v.at[c]],
                    rows_v.at[pl.ds(c * _CHUNK, _CHUNK)],
                    sem,
                )
            )
        for cp in copies:
            cp.wait()
        pltpu.sync_copy(rows_v, out_hbm.at[pl.ds(wid * rpw, rpw)])

    return gather_kernel(emb, ids3)


def _sc_transpose(embT):
    V = embT.shape[1]
    WP = 800
    mesh = plsc.VectorSubcoreMesh(core_axis_name="c", subcore_axis_name="s")

    @functools.partial(
        pl.kernel,
        mesh=mesh,
        out_type=jax.ShapeDtypeStruct((V, D), jnp.float32),
        scratch_types=[
            pltpu.VMEM((D, WP), jnp.float32),
            pltpu.VMEM((WP, D), jnp.float32),
        ],
        compiler_params=pltpu.CompilerParams(use_tc_tiling_on_sc=False,
                                             needs_layout_passes=False),
    )
    def tr_kernel(embT_hbm, out_hbm, buf, rows_v):
        wid = lax.axis_index("s") * _NC + lax.axis_index("c")
        iota = lax.broadcasted_iota(jnp.int32, (16,), 0)
        zero16 = jnp.zeros((16,), jnp.int32)

        def do_pass(base):
            pltpu.sync_copy(embT_hbm.at[:, pl.ds(base, WP)], buf)

            def body(i, carry):
                for u in range(2):
                    c = 2 * i + u
                    cvec = zero16 + c
                    for b in range(4):
                        v = plsc.load_gather(buf, [iota + 16 * b, cvec])
                        plsc.store_scatter(rows_v, [cvec, iota + 16 * b], v)
                return carry

            lax.fori_loop(0, WP // 2, body, 0)
            pltpu.sync_copy(rows_v, out_hbm.at[pl.ds(base, WP)])

        n_full = 31

        @pl.when(wid < n_full)
        def _():
            for p in range(4):
                do_pass(wid * 4 * WP + p * WP)

        @pl.when(wid == n_full)
        def _():
            do_pass(n_full * 4 * WP)

    return tr_kernel(embT)


_NB = 64


def _tc_body(a_ref, hid_ref,
             wc_ref, bc_ref, g_ref,
             wio_ref, bio_ref, wih_ref, bih_ref, whh_ref, bhh_ref,
             out_ref, cor_ref,
             h_sc, p_sc, x_sc):
    f32 = jnp.float32
    bf16 = jnp.bfloat16
    R = _NB * L

    def mm(x, w_ref):
        return jnp.dot(x, w_ref[...], preferred_element_type=f32)

    hid = hid_ref[...].astype(bf16)
    hk = jnp.tanh(mm(hid, wc_ref) + bc_ref[...])
    hk2 = (hk * hk).astype(bf16)
    ssq = jnp.dot(hk2, g_ref[...], preferred_element_type=f32)
    hk = hk * lax.rsqrt(ssq + 1e-12)
    for k in range(K):
        cor_ref[k] = jnp.reshape(hk[:, k * C:(k + 1) * C], (_NB, L, C))
    h_sc[...] = hk

    a_bf = a_ref[...].astype(bf16)
    col = lax.broadcasted_iota(jnp.int32, (L, 2 * D), 1)
    mlo = (col < D).astype(bf16)
    mhi = (col >= D).astype(bf16)
    for _ in range(ITER):
        h = h_sc[...]
        h_bf = h.astype(bf16)
        p_sc[...] = (mm(h_bf, wio_ref) + bio_ref[...]).astype(bf16)
        for s in range(_NB):
            p_s = p_sc[pl.ds(s * L, L), :]
            pp = jnp.concatenate([p_s * mlo, p_s * mhi], axis=0)
            x_sc[pl.ds(s * L, L), :] = jnp.dot(
                a_bf[s], pp, preferred_element_type=f32).astype(bf16)
        gi = mm(x_sc[...], wih_ref) + bih_ref[...]
        gh = mm(h_bf, whh_ref) + bhh_ref[...]
        r = jax.nn.sigmoid(gi[:, :D] + gh[:, :D])
        z = jax.nn.sigmoid(gi[:, D:2 * D] + gh[:, D:2 * D])
        n = jnp.tanh(gi[:, 2 * D:] + r * gh[:, 2 * D:])
        h_sc[...] = (1.0 - z) * n + z * h
    out_ref[...] = jnp.reshape(h_sc[...], (_NB, L, D))


def _block_diag(w):
    eye = jnp.eye(K, dtype=w.dtype)
    t = w[:, :, None, :] * eye[:, None, :, None]
    return t.reshape(K * w.shape[1], K * w.shape[2])


def _pack_weights(Wc, bc, Win, bin_, Wout, bout, Wih, bih, Whh, bhh):
    wc_all = Wc.transpose(1, 0, 2).reshape(D, K * C)
    bc2 = bc.reshape(1, K * C)
    gmask = jnp.kron(jnp.eye(K, dtype=jnp.float32),
                     jnp.ones((C, C), jnp.float32))
    wio = jnp.concatenate([_block_diag(Win), _block_diag(Wout)], axis=1)
    bio = jnp.concatenate([bin_.reshape(1, K * C), bout.reshape(1, K * C)],
                          axis=1)

    def gates(w):
        return jnp.concatenate(
            [_block_diag(w[:, :, g * C:(g + 1) * C]) for g in range(3)],
            axis=1)

    wih_p = jnp.concatenate([gates(Wih[:, :C, :]), gates(Wih[:, C:, :])],
                            axis=0)
    bih_p = jnp.concatenate(
        [bih[:, g * C:(g + 1) * C].reshape(1, K * C) for g in range(3)],
        axis=1)
    whh_p = gates(Whh)
    bhh_p = jnp.concatenate(
        [bhh[:, g * C:(g + 1) * C].reshape(1, K * C) for g in range(3)],
        axis=1)
    bf16 = jnp.bfloat16
    return (wc_all.astype(bf16), bc2, gmask.astype(bf16), wio.astype(bf16),
            bio, wih_p.astype(bf16), bih_p, whh_p.astype(bf16), bhh_p)


def _dense(A, gathered, packed):
    B = A.shape[0]
    R = _NB * L
    grid = B // _NB
    f32 = jnp.float32

    def wspec(shape):
        nd = len(shape)
        return pl.BlockSpec(shape, lambda i, _n=nd: (0,) * _n)

    in_specs = [
        pl.BlockSpec((_NB, L, 2 * L), lambda i: (i, 0, 0)),
        pl.BlockSpec((R, D), lambda i: (i, 0)),
    ] + [wspec(p.shape) for p in packed]

    out_specs = [
        pl.BlockSpec((_NB, L, D), lambda i: (i, 0, 0)),
        pl.BlockSpec((K, _NB, L, C), lambda i: (0, i, 0, 0)),
    ]

    out, cor = pl.pallas_call(
        _tc_body,
        grid=(grid,),
        in_specs=in_specs,
        out_specs=out_specs,
        out_shape=[
            jax.ShapeDtypeStruct((B, L, D), f32),
            jax.ShapeDtypeStruct((K, B, L, C), f32),
        ],
        scratch_shapes=[
            pltpu.VMEM((R, D), f32),
            pltpu.VMEM((R, 2 * D), jnp.bfloat16),
            pltpu.VMEM((R, 2 * D), jnp.bfloat16),
        ],
    )(A, gathered, *packed)
    return out, cor


def kernel(inputs, A, emb, Wc, bc, Win, bin_, Wout, bout, Wih, bih, Whh, bhh):
    B, Ls = inputs.shape
    rows = B * Ls
    rpw = rows // _NW
    n_chunk = rpw // _CHUNK
    ids3 = inputs.astype(jnp.int32).reshape(_NW, n_chunk, _CHUNK)
    emb_rm = _sc_transpose(emb.T)
    gathered = _sc_gather(emb_rm, ids3)
    packed = _pack_weights(Wc, bc, Win, bin_, Wout, bout, Wih, bih, Whh, bhh)
    return _dense(A, gathered, packed)


# speedup vs baseline: 1.4433x; 1.4433x over previous
import functools

import jax
import jax.numpy as jnp
from jax import lax
from jax.experimental import pallas as pl
from jax.experimental.pallas import tpu as pltpu
from jax.experimental.pallas import tpu_sc as plsc

D = 64
K = 4
C = 16
L = 50
ITER = 2

_NC = 2
_NS = 16
_NW = _NC * _NS
_CHUNK = 80


def _sc_gather(emb, ids3):
    n_chunk = ids3.shape[1]
    rpw = n_chunk * _CHUNK
    total = _NW * rpw
    mesh = plsc.VectorSubcoreMesh(core_axis_name="c", subcore_axis_name="s")

    @functools.partial(
        pl.kernel,
        mesh=mesh,
        out_type=jax.ShapeDtypeStruct((total, D), jnp.float32),
        scratch_types=[
            pltpu.VMEM((n_chunk, _CHUNK), jnp.int32),
            pltpu.VMEM((rpw, D), jnp.float32),
            pltpu.SemaphoreType.DMA,
        ],
        compiler_params=pltpu.CompilerParams(use_tc_tiling_on_sc=False),
    )
    def gather_kernel(emb_hbm, idx_hbm, out_hbm, idx_v, rows_v, sem):
        wid = lax.axis_index("s") * _NC + lax.axis_index("c")
        pltpu.sync_copy(idx_hbm.at[wid], idx_v)
        copies = []
        for c in range(n_chunk):
            copies.append(
                pltpu.async_copy(
                    emb_hbm.at[idx_v.at[c]],
                    rows_v.at[pl.ds(c * _CHUNK, _CHUNK)],
                    sem,
                )
            )
        for cp in copies:
            cp.wait()
        pltpu.sync_copy(rows_v, out_hbm.at[pl.ds(wid * rpw, rpw)])

    return gather_kernel(emb, ids3)


_NB = 64


def _tc_body(a_ref, hid_ref,
             wc_ref, bc_ref, g_ref,
             wio_ref, bio_ref, wih_ref, bih_ref, whh_ref, bhh_ref,
             out_ref, cor_ref,
             h_sc, p_sc, x_sc):
    f32 = jnp.float32
    bf16 = jnp.bfloat16
    R = _NB * L

    def mm(x, w_ref):
        return jnp.dot(x, w_ref[...], preferred_element_type=f32)

    hid = hid_ref[...].astype(bf16)
    hk = jnp.tanh(mm(hid, wc_ref) + bc_ref[...])
    hk2 = (hk * hk).astype(bf16)
    ssq = jnp.dot(hk2, g_ref[...], preferred_element_type=f32)
    hk = hk * lax.rsqrt(ssq + 1e-12)
    for k in range(K):
        cor_ref[k] = jnp.reshape(hk[:, k * C:(k + 1) * C], (_NB, L, C))
    h_sc[...] = hk

    a_bf = a_ref[...].astype(bf16)
    col = lax.broadcasted_iota(jnp.int32, (L, 2 * D), 1)
    mlo = (col < D).astype(bf16)
    mhi = (col >= D).astype(bf16)
    for _ in range(ITER):
        h = h_sc[...]
        h_bf = h.astype(bf16)
        p_sc[...] = (mm(h_bf, wio_ref) + bio_ref[...]).astype(bf16)
        for s in range(_NB):
            p_s = p_sc[pl.ds(s * L, L), :]
            pp = jnp.concatenate([p_s * mlo, p_s * mhi], axis=0)
            x_sc[pl.ds(s * L, L), :] = jnp.dot(
                a_bf[s], pp, preferred_element_type=f32).astype(bf16)
        gi = mm(x_sc[...], wih_ref) + bih_ref[...]
        gh = mm(h_bf, whh_ref) + bhh_ref[...]
        r = jax.nn.sigmoid(gi[:, :D] + gh[:, :D])
        z = jax.nn.sigmoid(gi[:, D:2 * D] + gh[:, D:2 * D])
        n = jnp.tanh(gi[:, 2 * D:] + r * gh[:, 2 * D:])
        h_sc[...] = (1.0 - z) * n + z * h
    out_ref[...] = jnp.reshape(h_sc[...], (_NB, L, D))


def _block_diag(w):
    eye = jnp.eye(K, dtype=w.dtype)
    t = w[:, :, None, :] * eye[:, None, :, None]
    return t.reshape(K * w.shape[1], K * w.shape[2])


def _pack_weights(Wc, bc, Win, bin_, Wout, bout, Wih, bih, Whh, bhh):
    wc_all = Wc.transpose(1, 0, 2).reshape(D, K * C)
    bc2 = bc.reshape(1, K * C)
    gmask = jnp.kron(jnp.eye(K, dtype=jnp.float32),
                     jnp.ones((C, C), jnp.float32))
    wio = jnp.concatenate([_block_diag(Win), _block_diag(Wout)], axis=1)
    bio = jnp.concatenate([bin_.reshape(1, K * C), bout.reshape(1, K * C)],
                          axis=1)

    def gates(w):
        return jnp.concatenate(
            [_block_diag(w[:, :, g * C:(g + 1) * C]) for g in range(3)],
            axis=1)

    wih_p = jnp.concatenate([gates(Wih[:, :C, :]), gates(Wih[:, C:, :])],
                            axis=0)
    bih_p = jnp.concatenate(
        [bih[:, g * C:(g + 1) * C].reshape(1, K * C) for g in range(3)],
        axis=1)
    whh_p = gates(Whh)
    bhh_p = jnp.concatenate(
        [bhh[:, g * C:(g + 1) * C].reshape(1, K * C) for g in range(3)],
        axis=1)
    bf16 = jnp.bfloat16
    return (wc_all.astype(bf16), bc2, gmask.astype(bf16), wio.astype(bf16),
            bio, wih_p.astype(bf16), bih_p, whh_p.astype(bf16), bhh_p)


def _dense(A, gathered, packed):
    B = A.shape[0]
    R = _NB * L
    grid = B // _NB
    f32 = jnp.float32

    def wspec(shape):
        nd = len(shape)
        return pl.BlockSpec(shape, lambda i, _n=nd: (0,) * _n)

    in_specs = [
        pl.BlockSpec((_NB, L, 2 * L), lambda i: (i, 0, 0)),
        pl.BlockSpec((R, D), lambda i: (i, 0)),
    ] + [wspec(p.shape) for p in packed]

    out_specs = [
        pl.BlockSpec((_NB, L, D), lambda i: (i, 0, 0)),
        pl.BlockSpec((K, _NB, L, C), lambda i: (0, i, 0, 0)),
    ]

    out, cor = pl.pallas_call(
        _tc_body,
        grid=(grid,),
        in_specs=in_specs,
        out_specs=out_specs,
        out_shape=[
            jax.ShapeDtypeStruct((B, L, D), f32),
            jax.ShapeDtypeStruct((K, B, L, C), f32),
        ],
        scratch_shapes=[
            pltpu.VMEM((R, D), f32),
            pltpu.VMEM((R, 2 * D), jnp.bfloat16),
            pltpu.VMEM((R, 2 * D), jnp.bfloat16),
        ],
    )(A, gathered, *packed)
    return out, cor


def kernel(inputs, A, emb, Wc, bc, Win, bin_, Wout, bout, Wih, bih, Whh, bhh):
    B, Ls = inputs.shape
    rows = B * Ls
    rpw = rows // _NW
    n_chunk = rpw // _CHUNK
    ids3 = inputs.astype(jnp.int32).reshape(_NW, n_chunk, _CHUNK)
    gathered = _sc_gather(emb, ids3)
    packed = _pack_weights(Wc, bc, Win, bin_, Wout, bout, Wih, bih, Whh, bhh)
    return _dense(A, gathered, packed)
